# baseline (speedup 1.0000x reference)
import functools

import jax
import jax.numpy as jnp
from jax import lax
from jax.experimental import pallas as pl
from jax.experimental.pallas import tpu as pltpu
from jax.experimental.pallas import tpu_sc as plsc

SEQ = 81
VOCAB = 10
HID = 512
CH = 64


def _comb_body(tok_ref, pos_ref, out_ref):
    pos = pos_ref[...]
    for v in range(VOCAB):
        out_ref[v] = pos + jnp.broadcast_to(tok_ref[v, :][None, :], (SEQ, HID))


def _build_comb(token_emb, pos_emb):
    comb3 = pl.pallas_call(
        _comb_body,
        out_shape=jax.ShapeDtypeStruct((VOCAB, SEQ, HID), jnp.float32),
    )(token_emb, pos_emb)
    return comb3.reshape(VOCAB * SEQ, HID)


def _sc_gather(x_flat, comb):
    BS = x_flat.shape[0]
    info = plsc.get_sparse_core_info()
    NC, NS = info.num_cores, info.num_subcores
    NW = NC * NS
    rpw = BS // NW
    nch = rpw // CH
    npair = nch // 2
    mesh = plsc.VectorSubcoreMesh(core_axis_name="c", subcore_axis_name="s")

    @functools.partial(
        pl.kernel,
        mesh=mesh,
        out_type=jax.ShapeDtypeStruct((BS, HID), jnp.float32),
        scratch_types=[
            pltpu.VMEM((rpw,), jnp.int32),
            pltpu.VMEM((CH,), jnp.int32),
            pltpu.VMEM((CH,), jnp.int32),
            pltpu.VMEM((CH, HID), jnp.float32),
            pltpu.VMEM((CH, HID), jnp.float32),
            pltpu.SemaphoreType.DMA,
            pltpu.SemaphoreType.DMA,
            pltpu.SemaphoreType.DMA,
        ],
    )
    def k(x_hbm, comb_hbm, out_hbm, xv, idx_a, idx_b, rows_a, rows_b,
          sem_x, sem_a, sem_b):
        wid = lax.axis_index("s") * NC + lax.axis_index("c")
        base0 = wid * rpw
        pltpu.make_async_copy(x_hbm.at[pl.ds(base0, rpw)], xv, sem_x).start()
        pltpu.make_async_copy(x_hbm.at[pl.ds(base0, rpw)], xv, sem_x).wait()

        def make_idx(g, idx_ref):
            cb = g * CH
            for j in range(CH // 16):
                xvec = xv[pl.ds(cb + j * 16, 16)]
                p = (base0 + cb + j * 16
                     + lax.broadcasted_iota(jnp.int32, (16,), 0))
                sv = lax.rem(p, SEQ)
                idx_ref[pl.ds(j * 16, 16)] = xvec * SEQ + sv

        def start_a():
            pltpu.make_async_copy(comb_hbm.at[idx_a], rows_a, sem_a).start()

        def start_b():
            pltpu.make_async_copy(comb_hbm.at[idx_b], rows_b, sem_b).start()

        make_idx(0, idx_a)
        start_a()

        def pair(h, _):
            g0 = 2 * h
            g1 = g0 + 1
            pltpu.make_async_copy(comb_hbm.at[idx_a], rows_a, sem_a).wait()
            make_idx(g1, idx_b)
            start_b()
            pltpu.sync_copy(rows_a, out_hbm.at[pl.ds(base0 + g0 * CH, CH)])
            pltpu.make_async_copy(comb_hbm.at[idx_b], rows_b, sem_b).wait()

            @pl.when(h + 1 < npair)
            def _next():
                make_idx(g1 + 1, idx_a)
                start_a()

            pltpu.sync_copy(rows_b, out_hbm.at[pl.ds(base0 + g1 * CH, CH)])
            return 0

        lax.fori_loop(0, npair, pair, 0)

    return k(x_flat, comb)


def kernel(x, token_emb, pos_emb):
    B = x.shape[0]
    comb = _build_comb(token_emb, pos_emb)
    x_flat = x.reshape(B * SEQ)
    out = _sc_gather(x_flat, comb)
    return out.reshape(B, SEQ, HID)


# speedup vs baseline: 1.9497x; 1.9497x over previous
import functools

import jax
import jax.numpy as jnp
from jax import lax
from jax.experimental import pallas as pl
from jax.experimental.pallas import tpu as pltpu

SEQ = 81
VOCAB = 10
HID = 512
NB = 32
NBUF = 8


def _compute(x, tok, pos):
    shape = (NB, SEQ, HID)
    xb = jnp.broadcast_to(x[:, :, None], shape)

    def tv(v):
        return jnp.broadcast_to(tok[v, :][None, None, :], shape)

    m0 = (xb & 1) != 0
    m1 = (xb & 2) != 0
    m2 = (xb & 4) != 0
    m3 = (xb & 8) != 0
    t01 = jnp.where(m0, tv(1), tv(0))
    t23 = jnp.where(m0, tv(3), tv(2))
    t45 = jnp.where(m0, tv(5), tv(4))
    t67 = jnp.where(m0, tv(7), tv(6))
    t89 = jnp.where(m0, tv(9), tv(8))
    t03 = jnp.where(m1, t23, t01)
    t47 = jnp.where(m1, t67, t45)
    t07 = jnp.where(m2, t47, t03)
    tok_sel = jnp.where(m3, t89, t07)
    return tok_sel + jnp.broadcast_to(pos[None, :, :], shape)


def _body(x_hbm, tok_ref, pos_ref, out_hbm, x_all, bufs, in_sem, out_sems):
    nblk = x_hbm.shape[0] // NB
    pltpu.make_async_copy(x_hbm, x_all, in_sem).start()
    pltpu.make_async_copy(x_hbm, x_all, in_sem).wait()
    tok = tok_ref[...]
    pos = pos_ref[...]

    def step(i, _):
        slot = lax.rem(i, NBUF)

        @pl.when(i >= NBUF)
        def _wait_prev():
            prev = i - NBUF
            pltpu.make_async_copy(
                bufs.at[slot],
                out_hbm.at[pl.ds(prev * NB, NB)],
                out_sems.at[slot],
            ).wait()

        x = x_all[pl.ds(i * NB, NB), :]
        bufs[slot] = _compute(x, tok, pos)
        pltpu.make_async_copy(
            bufs.at[slot],
            out_hbm.at[pl.ds(i * NB, NB)],
            out_sems.at[slot],
        ).start()
        return 0

    lax.fori_loop(0, nblk, step, 0)

    def drain(k, _):
        slot = lax.rem(nblk - NBUF + k, NBUF)
        pltpu.make_async_copy(
            bufs.at[slot],
            out_hbm.at[pl.ds((nblk - NBUF + k) * NB, NB)],
            out_sems.at[slot],
        ).wait()
        return 0

    lax.fori_loop(0, NBUF, drain, 0)


def kernel(x, token_emb, pos_emb):
    B = x.shape[0]
    out = pl.pallas_call(
        _body,
        in_specs=[
            pl.BlockSpec(memory_space=pl.ANY),
            pl.BlockSpec(memory_space=pltpu.MemorySpace.VMEM),
            pl.BlockSpec(memory_space=pltpu.MemorySpace.VMEM),
        ],
        out_specs=pl.BlockSpec(memory_space=pl.ANY),
        out_shape=jax.ShapeDtypeStruct((B, SEQ, HID), jnp.float32),
        scratch_shapes=[
            pltpu.VMEM((B, SEQ), jnp.int32),
            pltpu.VMEM((NBUF, NB, SEQ, HID), jnp.float32),
            pltpu.SemaphoreType.DMA,
            pltpu.SemaphoreType.DMA((NBUF,)),
        ],
    )(x, token_emb, pos_emb)
    return out


# speedup vs baseline: 1.9535x; 1.0020x over previous
import functools

import jax
import jax.numpy as jnp
from jax import lax
from jax.experimental import pallas as pl
from jax.experimental.pallas import tpu as pltpu

SEQ = 81
VOCAB = 10
HID = 512
NB = 64
NBUF = 4


def _compute(x, tok, pos):
    shape = (NB, SEQ, HID)
    xb = jnp.broadcast_to(x[:, :, None], shape)

    def tv(v):
        return jnp.broadcast_to(tok[v, :][None, None, :], shape)

    m0 = (xb & 1) != 0
    m1 = (xb & 2) != 0
    m2 = (xb & 4) != 0
    m3 = (xb & 8) != 0
    t01 = jnp.where(m0, tv(1), tv(0))
    t23 = jnp.where(m0, tv(3), tv(2))
    t45 = jnp.where(m0, tv(5), tv(4))
    t67 = jnp.where(m0, tv(7), tv(6))
    t89 = jnp.where(m0, tv(9), tv(8))
    t03 = jnp.where(m1, t23, t01)
    t47 = jnp.where(m1, t67, t45)
    t07 = jnp.where(m2, t47, t03)
    tok_sel = jnp.where(m3, t89, t07)
    return tok_sel + jnp.broadcast_to(pos[None, :, :], shape)


def _body(x_hbm, tok_ref, pos_ref, out_hbm, x_all, b0, b1, b2, b3,
          in_sem, out_sems):
    nblk = x_hbm.shape[0] // NB
    nround = nblk // NBUF
    slots = (b0, b1, b2, b3)
    pltpu.make_async_copy(x_hbm, x_all, in_sem).start()
    pltpu.make_async_copy(x_hbm, x_all, in_sem).wait()
    tok = tok_ref[...]
    pos = pos_ref[...]

    def round_(r, _):
        for s in range(NBUF):
            i = r * NBUF + s
            buf = slots[s]

            @pl.when(r > 0)
            def _wait_prev():
                pltpu.make_async_copy(
                    buf,
                    out_hbm.at[pl.ds((i - NBUF) * NB, NB)],
                    out_sems.at[s],
                ).wait()

            buf[...] = _compute(x_all[pl.ds(i * NB, NB), :], tok, pos)
            pltpu.make_async_copy(
                buf,
                out_hbm.at[pl.ds(i * NB, NB)],
                out_sems.at[s],
            ).start()
        return 0

    lax.fori_loop(0, nround, round_, 0)

    for s in range(NBUF):
        i = (nround - 1) * NBUF + s
        pltpu.make_async_copy(
            slots[s],
            out_hbm.at[pl.ds(i * NB, NB)],
            out_sems.at[s],
        ).wait()


def kernel(x, token_emb, pos_emb):
    B = x.shape[0]
    out = pl.pallas_call(
        _body,
        in_specs=[
            pl.BlockSpec(memory_space=pl.ANY),
            pl.BlockSpec(memory_space=pltpu.MemorySpace.VMEM),
            pl.BlockSpec(memory_space=pltpu.MemorySpace.VMEM),
        ],
        out_specs=pl.BlockSpec(memory_space=pl.ANY),
        out_shape=jax.ShapeDtypeStruct((B, SEQ, HID), jnp.float32),
        scratch_shapes=[
            pltpu.VMEM((B, SEQ), jnp.int32),
            pltpu.VMEM((NB, SEQ, HID), jnp.float32),
            pltpu.VMEM((NB, SEQ, HID), jnp.float32),
            pltpu.VMEM((NB, SEQ, HID), jnp.float32),
            pltpu.VMEM((NB, SEQ, HID), jnp.float32),
            pltpu.SemaphoreType.DMA,
            pltpu.SemaphoreType.DMA((NBUF,)),
        ],
    )(x, token_emb, pos_emb)
    return out
